# Initial kernel scaffold; baseline (speedup 1.0000x reference)
import jax
import jax.numpy as jnp
from jax.experimental import pallas as pl


def kernel(x, edge_index, edge_type, r, W, b):
    raise NotImplementedError("write your pallas kernel here")


# speedup vs baseline: 4.7763x; 4.7763x over previous
import functools

import jax
import jax.numpy as jnp
from jax import lax
from jax.experimental import pallas as pl
from jax.experimental.pallas import tpu as pltpu
from jax.experimental.pallas import tpu_sc as plsc

_LANES = 16
_BATCH = 80
_CNT_W = 128
_ZCH = 8


def _sc_segment_sum(x, src, dst, n_acc, batches_per_tile):
  info = plsc.get_sparse_core_info()
  nc, ns = info.num_cores, info.num_subcores
  d = x.shape[1]
  rows_per_tile = n_acc // ns
  mesh = plsc.VectorSubcoreMesh(core_axis_name="c", subcore_axis_name="s")

  @functools.partial(
      pl.kernel,
      out_type=(
          jax.ShapeDtypeStruct((nc, n_acc, d), jnp.float32),
      ),
      mesh=mesh,
      scratch_types=dict(
          srcb=pltpu.VMEM((_BATCH,), jnp.int32),
          dstb=pltpu.VMEM((_BATCH,), jnp.int32),
          rows_v=pltpu.VMEM((_BATCH, d), jnp.float32),
          acc_sh=pltpu.VMEM_SHARED((n_acc, d), jnp.float32),
          sem=pltpu.SemaphoreType.DMA,
      ),
  )
  def k(x_hbm, src_hbm, dst_hbm, acc_out, srcb, dstb, rows_v, acc_sh, sem):
    cid = lax.axis_index("c")
    sid = lax.axis_index("s")
    wid = cid * ns + sid

    zeros16 = jnp.zeros((_LANES,), jnp.float32)

    def fill_rows(i, _):
      r = i // (d // _LANES)
      c = (i % (d // _LANES)) * _LANES
      rows_v[r, pl.ds(c, _LANES)] = zeros16
      return 0
    lax.fori_loop(0, _ZCH * (d // _LANES), fill_rows, 0)

    zbase = sid * rows_per_tile

    def zero_acc(t, _):
      pltpu.sync_copy(rows_v.at[pl.ds(0, _ZCH)],
                      acc_sh.at[pl.ds(zbase + t * _ZCH, _ZCH)])
      return 0
    lax.fori_loop(0, rows_per_tile // _ZCH, zero_acc, 0)

    plsc.subcore_barrier()

    def edge_body(j, _):
      base = (wid * batches_per_tile + j) * _BATCH
      pltpu.sync_copy(src_hbm.at[pl.ds(base, _BATCH)], srcb)
      pltpu.sync_copy(dst_hbm.at[pl.ds(base, _BATCH)], dstb)
      pltpu.async_copy(x_hbm.at[srcb], rows_v, sem).wait()
      pltpu.sync_copy(rows_v, acc_sh.at[dstb], add=True)
      return 0
    lax.fori_loop(0, batches_per_tile, edge_body, 0)
    plsc.subcore_barrier()

    pltpu.sync_copy(acc_sh.at[pl.ds(zbase, rows_per_tile)],
                    acc_out.at[cid, pl.ds(zbase, rows_per_tile)])

  return k(x, src, dst)


def _sc_counts(dst, n_acc, batches_per_tile):
  info = plsc.get_sparse_core_info()
  nc, ns = info.num_cores, info.num_subcores
  rows_per_tile = n_acc // ns
  mesh = plsc.VectorSubcoreMesh(core_axis_name="c", subcore_axis_name="s")

  @functools.partial(
      pl.kernel,
      out_type=(jax.ShapeDtypeStruct((nc, n_acc, _CNT_W), jnp.float32),),
      mesh=mesh,
      scratch_types=dict(
          dstb=pltpu.VMEM((_BATCH,), jnp.int32),
          ones_v=pltpu.VMEM((_BATCH, _CNT_W), jnp.float32),
          zcnt_v=pltpu.VMEM((_ZCH, _CNT_W), jnp.float32),
          cnt_sh=pltpu.VMEM_SHARED((n_acc, _CNT_W), jnp.float32),
      ),
  )
  def k(dst_hbm, cnt_out, dstb, ones_v, zcnt_v, cnt_sh):
    cid = lax.axis_index("c")
    sid = lax.axis_index("s")
    wid = cid * ns + sid

    ones16 = jnp.ones((_LANES,), jnp.float32)
    zeros16 = jnp.zeros((_LANES,), jnp.float32)
    lanes_per_row = _CNT_W // _LANES

    def fill_ones(i, _):
      r = i // lanes_per_row
      c = (i % lanes_per_row) * _LANES
      ones_v[r, pl.ds(c, _LANES)] = ones16
      return 0
    lax.fori_loop(0, _BATCH * lanes_per_row, fill_ones, 0)

    def fill_zeros(i, _):
      r = i // lanes_per_row
      c = (i % lanes_per_row) * _LANES
      zcnt_v[r, pl.ds(c, _LANES)] = zeros16
      return 0
    lax.fori_loop(0, _ZCH * lanes_per_row, fill_zeros, 0)
    zbase = sid * rows_per_tile

    def zero_cnt(t, _):
      pltpu.sync_copy(zcnt_v, cnt_sh.at[pl.ds(zbase + t * _ZCH, _ZCH)])
      return 0
    lax.fori_loop(0, rows_per_tile // _ZCH, zero_cnt, 0)
    plsc.subcore_barrier()

    def edge_body(j, _):
      base = (wid * batches_per_tile + j) * _BATCH
      pltpu.sync_copy(dst_hbm.at[pl.ds(base, _BATCH)], dstb)
      pltpu.sync_copy(ones_v, cnt_sh.at[dstb], add=True)
      return 0
    lax.fori_loop(0, batches_per_tile, edge_body, 0)
    plsc.subcore_barrier()

    pltpu.sync_copy(cnt_sh.at[pl.ds(zbase, rows_per_tile)],
                    cnt_out.at[cid, pl.ds(zbase, rows_per_tile)])

  return k(dst)


def _tc_body(x_ref, acc_ref, cnt_ref, w_ref, b_ref, o_ref):
  d = x_ref.shape[1]
  s = acc_ref[0] + acc_ref[1]
  cnt = cnt_ref[0, :, 0:1] + cnt_ref[1, :, 0:1]
  inv = 1.0 / jnp.maximum(cnt, 1.0)
  z = (jnp.dot(x_ref[...], w_ref[0:d, :], preferred_element_type=jnp.float32)
       + jnp.dot(s, w_ref[d:2 * d, :], preferred_element_type=jnp.float32)
       * inv + b_ref[...])
  o_ref[...] = jnp.where(z > 0, z, 0.01 * z)


def kernel(x, edge_index, edge_type, r, W, b):
  del edge_type, r
  n, d = x.shape
  e = edge_index.shape[1]
  n_tiles = 32
  per_tile_edges = -(-e // (n_tiles * _BATCH)) * _BATCH
  ep = per_tile_edges * n_tiles
  batches_per_tile = per_tile_edges // _BATCH
  n_acc = -(-n // 128) * 128
  pad_rows = max(n_acc - n, 1)

  pad = ep - e
  src = jnp.concatenate(
      [edge_index[0], jnp.arange(pad, dtype=jnp.int32) % n])
  dst = jnp.concatenate(
      [edge_index[1], n + jnp.arange(pad, dtype=jnp.int32) % pad_rows])

  (acc_p,) = _sc_segment_sum(x, src, dst, n_acc, batches_per_tile)
  token = lax.convert_element_type(acc_p[0, 0, 0], jnp.int32) * 0
  (cnt_p,) = _sc_counts(dst + token, n_acc, batches_per_tile)

  blk = 1000
  grid = (n // blk,)
  out = pl.pallas_call(
      _tc_body,
      grid=grid,
      in_specs=[
          pl.BlockSpec((blk, d), lambda i: (i, 0)),
          pl.BlockSpec((2, blk, d), lambda i: (0, i, 0)),
          pl.BlockSpec((2, blk, _CNT_W), lambda i: (0, i, 0)),
          pl.BlockSpec((2 * d, d), lambda i: (0, 0)),
          pl.BlockSpec((1, d), lambda i: (0, 0)),
      ],
      out_specs=pl.BlockSpec((blk, d), lambda i: (i, 0)),
      out_shape=jax.ShapeDtypeStruct((n, d), jnp.float32),
  )(x, acc_p, cnt_p, W, b.reshape(1, d))
  return out


# speedup vs baseline: 6.4854x; 1.3578x over previous
import functools

import jax
import jax.numpy as jnp
from jax import lax
from jax.experimental import pallas as pl
from jax.experimental.pallas import tpu as pltpu
from jax.experimental.pallas import tpu_sc as plsc

_LANES = 16
_BATCH = 80
_CNT_W = 128
_ZCH = 8


def _sc_segment_sum(x, src, dst, n_acc, batches_per_tile):
  info = plsc.get_sparse_core_info()
  nc, ns = info.num_cores, info.num_subcores
  d = x.shape[1]
  rows_per_tile = n_acc // ns
  mesh = plsc.VectorSubcoreMesh(core_axis_name="c", subcore_axis_name="s")

  @functools.partial(
      pl.kernel,
      out_type=(
          jax.ShapeDtypeStruct((nc, n_acc, d), jnp.float32),
      ),
      mesh=mesh,
      scratch_types=dict(
          srcb=pltpu.VMEM((_BATCH,), jnp.int32),
          dstb=pltpu.VMEM((_BATCH,), jnp.int32),
          srcb1=pltpu.VMEM((_BATCH,), jnp.int32),
          dstb1=pltpu.VMEM((_BATCH,), jnp.int32),
          rows_v=pltpu.VMEM((_BATCH, d), jnp.float32),
          rows_v1=pltpu.VMEM((_BATCH, d), jnp.float32),
          acc_sh=pltpu.VMEM_SHARED((n_acc, d), jnp.float32),
          sem=pltpu.SemaphoreType.DMA,
          sem1=pltpu.SemaphoreType.DMA,
      ),
  )
  def k(x_hbm, src_hbm, dst_hbm, acc_out, srcb, dstb, srcb1, dstb1,
        rows_v, rows_v1, acc_sh, sem, sem1):
    cid = lax.axis_index("c")
    sid = lax.axis_index("s")
    wid = cid * ns + sid

    zeros16 = jnp.zeros((_LANES,), jnp.float32)

    def fill_rows(i, _):
      r = i // (d // _LANES)
      c = (i % (d // _LANES)) * _LANES
      rows_v[r, pl.ds(c, _LANES)] = zeros16
      return 0
    lax.fori_loop(0, _ZCH * (d // _LANES), fill_rows, 0)

    zbase = sid * rows_per_tile

    def zero_acc(t, _):
      pltpu.sync_copy(rows_v.at[pl.ds(0, _ZCH)],
                      acc_sh.at[pl.ds(zbase + t * _ZCH, _ZCH)])
      return 0
    lax.fori_loop(0, rows_per_tile // _ZCH, zero_acc, 0)

    plsc.subcore_barrier()

    tbase = wid * batches_per_tile

    def stage(j, sb, db):
      base = (tbase + j) * _BATCH
      pltpu.sync_copy(src_hbm.at[pl.ds(base, _BATCH)], sb)
      pltpu.sync_copy(dst_hbm.at[pl.ds(base, _BATCH)], db)

    stage(0, srcb, dstb)
    cp0 = pltpu.async_copy(x_hbm.at[srcb], rows_v, sem)

    def pair_body(p, _):
      j0 = 2 * p
      stage(j0 + 1, srcb1, dstb1)
      pltpu.async_copy(x_hbm.at[srcb1], rows_v1, sem1)
      pltpu.make_async_copy(x_hbm.at[srcb], rows_v, sem).wait()
      pltpu.sync_copy(rows_v, acc_sh.at[dstb], add=True)
      @pl.when(j0 + 2 < batches_per_tile)
      def _():
        stage(j0 + 2, srcb, dstb)
        pltpu.async_copy(x_hbm.at[srcb], rows_v, sem)
      pltpu.make_async_copy(x_hbm.at[srcb1], rows_v1, sem1).wait()
      pltpu.sync_copy(rows_v1, acc_sh.at[dstb1], add=True)
      return 0
    lax.fori_loop(0, batches_per_tile // 2, pair_body, 0)

    if batches_per_tile % 2:
      pltpu.make_async_copy(x_hbm.at[srcb], rows_v, sem).wait()
      pltpu.sync_copy(rows_v, acc_sh.at[dstb], add=True)
    del cp0
    plsc.subcore_barrier()

    pltpu.sync_copy(acc_sh.at[pl.ds(zbase, rows_per_tile)],
                    acc_out.at[cid, pl.ds(zbase, rows_per_tile)])

  return k(x, src, dst)


def _sc_counts(dst, n_acc, batches_per_tile):
  info = plsc.get_sparse_core_info()
  nc, ns = info.num_cores, info.num_subcores
  rows_per_tile = n_acc // ns
  mesh = plsc.VectorSubcoreMesh(core_axis_name="c", subcore_axis_name="s")

  @functools.partial(
      pl.kernel,
      out_type=(jax.ShapeDtypeStruct((nc, n_acc, _CNT_W), jnp.float32),),
      mesh=mesh,
      scratch_types=dict(
          dstb=pltpu.VMEM((_BATCH,), jnp.int32),
          ones_v=pltpu.VMEM((_BATCH, _CNT_W), jnp.float32),
          zcnt_v=pltpu.VMEM((_ZCH, _CNT_W), jnp.float32),
          cnt_sh=pltpu.VMEM_SHARED((n_acc, _CNT_W), jnp.float32),
      ),
  )
  def k(dst_hbm, cnt_out, dstb, ones_v, zcnt_v, cnt_sh):
    cid = lax.axis_index("c")
    sid = lax.axis_index("s")
    wid = cid * ns + sid

    ones16 = jnp.ones((_LANES,), jnp.float32)
    zeros16 = jnp.zeros((_LANES,), jnp.float32)
    lanes_per_row = _CNT_W // _LANES

    def fill_ones(i, _):
      r = i // lanes_per_row
      c = (i % lanes_per_row) * _LANES
      ones_v[r, pl.ds(c, _LANES)] = ones16
      return 0
    lax.fori_loop(0, _BATCH * lanes_per_row, fill_ones, 0)

    def fill_zeros(i, _):
      r = i // lanes_per_row
      c = (i % lanes_per_row) * _LANES
      zcnt_v[r, pl.ds(c, _LANES)] = zeros16
      return 0
    lax.fori_loop(0, _ZCH * lanes_per_row, fill_zeros, 0)
    zbase = sid * rows_per_tile

    def zero_cnt(t, _):
      pltpu.sync_copy(zcnt_v, cnt_sh.at[pl.ds(zbase + t * _ZCH, _ZCH)])
      return 0
    lax.fori_loop(0, rows_per_tile // _ZCH, zero_cnt, 0)
    plsc.subcore_barrier()

    def edge_body(j, _):
      base = (wid * batches_per_tile + j) * _BATCH
      pltpu.sync_copy(dst_hbm.at[pl.ds(base, _BATCH)], dstb)
      pltpu.sync_copy(ones_v, cnt_sh.at[dstb], add=True)
      return 0
    lax.fori_loop(0, batches_per_tile, edge_body, 0)
    plsc.subcore_barrier()

    pltpu.sync_copy(cnt_sh.at[pl.ds(zbase, rows_per_tile)],
                    cnt_out.at[cid, pl.ds(zbase, rows_per_tile)])

  return k(dst)


def _tc_body(x_ref, acc_ref, cnt_ref, w_ref, b_ref, o_ref):
  d = x_ref.shape[1]
  s = acc_ref[0] + acc_ref[1]
  cnt = cnt_ref[0, :, 0:1] + cnt_ref[1, :, 0:1]
  inv = 1.0 / jnp.maximum(cnt, 1.0)
  z = (jnp.dot(x_ref[...], w_ref[0:d, :], preferred_element_type=jnp.float32)
       + jnp.dot(s, w_ref[d:2 * d, :], preferred_element_type=jnp.float32)
       * inv + b_ref[...])
  o_ref[...] = jnp.where(z > 0, z, 0.01 * z)


def kernel(x, edge_index, edge_type, r, W, b):
  del edge_type, r
  n, d = x.shape
  e = edge_index.shape[1]
  n_tiles = 32
  per_tile_edges = -(-e // (n_tiles * _BATCH)) * _BATCH
  ep = per_tile_edges * n_tiles
  batches_per_tile = per_tile_edges // _BATCH
  n_acc = -(-n // 128) * 128
  pad_rows = max(n_acc - n, 1)

  pad = ep - e
  src = jnp.concatenate(
      [edge_index[0], jnp.arange(pad, dtype=jnp.int32) % n])
  dst = jnp.concatenate(
      [edge_index[1], n + jnp.arange(pad, dtype=jnp.int32) % pad_rows])

  (acc_p,) = _sc_segment_sum(x, src, dst, n_acc, batches_per_tile)
  token = lax.convert_element_type(acc_p[0, 0, 0], jnp.int32) * 0
  (cnt_p,) = _sc_counts(dst + token, n_acc, batches_per_tile)

  blk = 1000
  grid = (n // blk,)
  out = pl.pallas_call(
      _tc_body,
      grid=grid,
      in_specs=[
          pl.BlockSpec((blk, d), lambda i: (i, 0)),
          pl.BlockSpec((2, blk, d), lambda i: (0, i, 0)),
          pl.BlockSpec((2, blk, _CNT_W), lambda i: (0, i, 0)),
          pl.BlockSpec((2 * d, d), lambda i: (0, 0)),
          pl.BlockSpec((1, d), lambda i: (0, 0)),
      ],
      out_specs=pl.BlockSpec((blk, d), lambda i: (i, 0)),
      out_shape=jax.ShapeDtypeStruct((n, d), jnp.float32),
  )(x, acc_p, cnt_p, W, b.reshape(1, d))
  return out


# speedup vs baseline: 7.5747x; 1.1680x over previous
import functools

import jax
import jax.numpy as jnp
from jax import lax
from jax.experimental import pallas as pl
from jax.experimental.pallas import tpu as pltpu
from jax.experimental.pallas import tpu_sc as plsc

_LANES = 16
_BATCH = 80
_CNT_W = 128
_ZCH = 8


def _sc_segment_sum(x, src, dst, n_acc, batches_per_tile):
  info = plsc.get_sparse_core_info()
  nc, ns = info.num_cores, info.num_subcores
  d = x.shape[1]
  rows_per_tile = n_acc // ns
  mesh = plsc.VectorSubcoreMesh(core_axis_name="c", subcore_axis_name="s")

  @functools.partial(
      pl.kernel,
      out_type=(
          jax.ShapeDtypeStruct((nc, n_acc, d), jnp.float32),
      ),
      mesh=mesh,
      scratch_types=dict(
          srcb=pltpu.VMEM((_BATCH,), jnp.int32),
          dstb=pltpu.VMEM((_BATCH,), jnp.int32),
          srcb1=pltpu.VMEM((_BATCH,), jnp.int32),
          dstb1=pltpu.VMEM((_BATCH,), jnp.int32),
          rows_v=pltpu.VMEM((_BATCH, d), jnp.float32),
          rows_v1=pltpu.VMEM((_BATCH, d), jnp.float32),
          acc_sh=pltpu.VMEM_SHARED((n_acc, d), jnp.float32),
          sem=pltpu.SemaphoreType.DMA,
          sem1=pltpu.SemaphoreType.DMA,
      ),
  )
  def k(x_hbm, src_hbm, dst_hbm, acc_out, srcb, dstb, srcb1, dstb1,
        rows_v, rows_v1, acc_sh, sem, sem1):
    cid = lax.axis_index("c")
    sid = lax.axis_index("s")
    wid = cid * ns + sid

    zeros16 = jnp.zeros((_LANES,), jnp.float32)

    def fill_rows(i, _):
      r = i // (d // _LANES)
      c = (i % (d // _LANES)) * _LANES
      rows_v[r, pl.ds(c, _LANES)] = zeros16
      return 0
    lax.fori_loop(0, _ZCH * (d // _LANES), fill_rows, 0)

    zbase = sid * rows_per_tile

    def zero_acc(t, _):
      pltpu.sync_copy(rows_v.at[pl.ds(0, _ZCH)],
                      acc_sh.at[pl.ds(zbase + t * _ZCH, _ZCH)])
      return 0
    lax.fori_loop(0, rows_per_tile // _ZCH, zero_acc, 0)

    plsc.subcore_barrier()

    tbase = wid * batches_per_tile

    def stage(j, sb, db):
      base = (tbase + j) * _BATCH
      pltpu.sync_copy(src_hbm.at[pl.ds(base, _BATCH)], sb)
      pltpu.sync_copy(dst_hbm.at[pl.ds(base, _BATCH)], db)

    stage(0, srcb, dstb)
    cp0 = pltpu.async_copy(x_hbm.at[srcb], rows_v, sem)

    def pair_body(p, _):
      j0 = 2 * p
      stage(j0 + 1, srcb1, dstb1)
      pltpu.async_copy(x_hbm.at[srcb1], rows_v1, sem1)
      pltpu.make_async_copy(x_hbm.at[srcb], rows_v, sem).wait()
      pltpu.sync_copy(rows_v, acc_sh.at[dstb], add=True)
      @pl.when(j0 + 2 < batches_per_tile)
      def _():
        stage(j0 + 2, srcb, dstb)
        pltpu.async_copy(x_hbm.at[srcb], rows_v, sem)
      pltpu.make_async_copy(x_hbm.at[srcb1], rows_v1, sem1).wait()
      pltpu.sync_copy(rows_v1, acc_sh.at[dstb1], add=True)
      return 0
    lax.fori_loop(0, batches_per_tile // 2, pair_body, 0)

    if batches_per_tile % 2:
      pltpu.make_async_copy(x_hbm.at[srcb], rows_v, sem).wait()
      pltpu.sync_copy(rows_v, acc_sh.at[dstb], add=True)
    del cp0
    plsc.subcore_barrier()

    pltpu.sync_copy(acc_sh.at[pl.ds(zbase, rows_per_tile)],
                    acc_out.at[cid, pl.ds(zbase, rows_per_tile)])

  return k(x, src, dst)


def _sc_counts(dst, n_acc, batches_per_tile):
  info = plsc.get_sparse_core_info()
  nc, ns = info.num_cores, info.num_subcores
  rows_per_tile = n_acc // ns
  mesh = plsc.VectorSubcoreMesh(core_axis_name="c", subcore_axis_name="s")

  @functools.partial(
      pl.kernel,
      out_type=(jax.ShapeDtypeStruct((nc, n_acc, _CNT_W), jnp.float32),),
      mesh=mesh,
      scratch_types=dict(
          dstb=pltpu.VMEM((_BATCH,), jnp.int32),
          dstb1=pltpu.VMEM((_BATCH,), jnp.int32),
          ones_v=pltpu.VMEM((_BATCH, _CNT_W), jnp.float32),
          zcnt_v=pltpu.VMEM((_ZCH, _CNT_W), jnp.float32),
          cnt_sh=pltpu.VMEM_SHARED((n_acc, _CNT_W), jnp.float32),
          semc=pltpu.SemaphoreType.DMA,
          semc1=pltpu.SemaphoreType.DMA,
      ),
  )
  def k(dst_hbm, cnt_out, dstb, dstb1, ones_v, zcnt_v, cnt_sh, semc, semc1):
    cid = lax.axis_index("c")
    sid = lax.axis_index("s")
    wid = cid * ns + sid

    ones16 = jnp.ones((_LANES,), jnp.float32)
    zeros16 = jnp.zeros((_LANES,), jnp.float32)
    lanes_per_row = _CNT_W // _LANES

    def fill_ones(i, _):
      r = i // lanes_per_row
      c = (i % lanes_per_row) * _LANES
      ones_v[r, pl.ds(c, _LANES)] = ones16
      return 0
    lax.fori_loop(0, _BATCH * lanes_per_row, fill_ones, 0)

    def fill_zeros(i, _):
      r = i // lanes_per_row
      c = (i % lanes_per_row) * _LANES
      zcnt_v[r, pl.ds(c, _LANES)] = zeros16
      return 0
    lax.fori_loop(0, _ZCH * lanes_per_row, fill_zeros, 0)
    zbase = sid * rows_per_tile

    def zero_cnt(t, _):
      pltpu.sync_copy(zcnt_v, cnt_sh.at[pl.ds(zbase + t * _ZCH, _ZCH)])
      return 0
    lax.fori_loop(0, rows_per_tile // _ZCH, zero_cnt, 0)
    plsc.subcore_barrier()

    tbase = wid * batches_per_tile

    def stage(j, db):
      pltpu.sync_copy(dst_hbm.at[pl.ds((tbase + j) * _BATCH, _BATCH)], db)

    stage(0, dstb)

    def pair_body(p, _):
      j0 = 2 * p
      pltpu.async_copy(ones_v, cnt_sh.at[dstb], semc, add=True)
      stage(j0 + 1, dstb1)
      pltpu.make_async_copy(ones_v, cnt_sh.at[dstb], semc).wait()
      pltpu.async_copy(ones_v, cnt_sh.at[dstb1], semc1, add=True)
      @pl.when(j0 + 2 < batches_per_tile)
      def _():
        stage(j0 + 2, dstb)
      pltpu.make_async_copy(ones_v, cnt_sh.at[dstb1], semc1).wait()
      return 0
    lax.fori_loop(0, batches_per_tile // 2, pair_body, 0)

    if batches_per_tile % 2:
      pltpu.sync_copy(ones_v, cnt_sh.at[dstb], add=True)
    plsc.subcore_barrier()

    pltpu.sync_copy(cnt_sh.at[pl.ds(zbase, rows_per_tile)],
                    cnt_out.at[cid, pl.ds(zbase, rows_per_tile)])

  return k(dst)


def _tc_body(x_ref, acc_ref, cnt_ref, w_ref, b_ref, o_ref):
  d = x_ref.shape[1]
  s = acc_ref[0] + acc_ref[1]
  cnt = cnt_ref[0, :, 0:1] + cnt_ref[1, :, 0:1]
  inv = 1.0 / jnp.maximum(cnt, 1.0)
  z = (jnp.dot(x_ref[...], w_ref[0:d, :], preferred_element_type=jnp.float32)
       + jnp.dot(s, w_ref[d:2 * d, :], preferred_element_type=jnp.float32)
       * inv + b_ref[...])
  o_ref[...] = jnp.where(z > 0, z, 0.01 * z)


def kernel(x, edge_index, edge_type, r, W, b):
  del edge_type, r
  n, d = x.shape
  e = edge_index.shape[1]
  n_tiles = 32
  per_tile_edges = -(-e // (n_tiles * _BATCH)) * _BATCH
  ep = per_tile_edges * n_tiles
  batches_per_tile = per_tile_edges // _BATCH
  n_acc = -(-n // 128) * 128
  pad_rows = max(n_acc - n, 1)

  pad = ep - e
  src = jnp.concatenate(
      [edge_index[0], jnp.arange(pad, dtype=jnp.int32) % n])
  dst = jnp.concatenate(
      [edge_index[1], n + jnp.arange(pad, dtype=jnp.int32) % pad_rows])

  (acc_p,) = _sc_segment_sum(x, src, dst, n_acc, batches_per_tile)
  token = lax.convert_element_type(acc_p[0, 0, 0], jnp.int32) * 0
  (cnt_p,) = _sc_counts(dst + token, n_acc, batches_per_tile)

  blk = 1000
  grid = (n // blk,)
  out = pl.pallas_call(
      _tc_body,
      grid=grid,
      in_specs=[
          pl.BlockSpec((blk, d), lambda i: (i, 0)),
          pl.BlockSpec((2, blk, d), lambda i: (0, i, 0)),
          pl.BlockSpec((2, blk, _CNT_W), lambda i: (0, i, 0)),
          pl.BlockSpec((2 * d, d), lambda i: (0, 0)),
          pl.BlockSpec((1, d), lambda i: (0, 0)),
      ],
      out_specs=pl.BlockSpec((blk, d), lambda i: (i, 0)),
      out_shape=jax.ShapeDtypeStruct((n, d), jnp.float32),
  )(x, acc_p, cnt_p, W, b.reshape(1, d))
  return out
